# baseline (speedup 1.0000x reference)
import functools

import jax
import jax.numpy as jnp
from jax.experimental import pallas as pl
from jax.experimental.pallas import tpu as pltpu


def _enc_topk_body(x_ref, pb_ref, lb_ref, w_ref,
                   pa_ref, tv_ref, ti_ref, fl_ref,
                   cm_ref, cs_ref, *, K, T, nh):
    h = pl.program_id(1)
    Bt = x_ref.shape[0]
    h_tile = w_ref.shape[0]
    C = min(128, h_tile)
    ns = h_tile // C
    NEG = jnp.float32(-jnp.inf)

    xc = x_ref[...] - pb_ref[...]
    pa_ref[...] = jax.lax.dot_general(
        xc, w_ref[...], (((1,), (1,)), ((), ())),
        preferred_element_type=jnp.float32) + lb_ref[...]

    @pl.when(h == 0)
    def _():
        cm_ref[...] = jnp.full(cm_ref.shape, NEG, jnp.float32)
        cs_ref[...] = jnp.zeros(cs_ref.shape, jnp.int32)

    mreg = [cm_ref[:, j * C:(j + 1) * C] for j in range(T)]
    sreg = [cs_ref[:, j * C:(j + 1) * C] for j in range(T)]
    for s in range(ns):
        v = pa_ref[:, s * C:(s + 1) * C]
        ci = jnp.zeros((Bt, C), jnp.int32) + (h * ns + s)
        for j in range(T):
            beat = v > mreg[j]
            mo, so = mreg[j], sreg[j]
            mreg[j] = jnp.where(beat, v, mo)
            sreg[j] = jnp.where(beat, ci, so)
            v = jnp.where(beat, mo, v)
            ci = jnp.where(beat, so, ci)
    for j in range(T):
        cm_ref[:, j * C:(j + 1) * C] = mreg[j]
        cs_ref[:, j * C:(j + 1) * C] = sreg[j]

    @pl.when(h == nh - 1)
    def _finish():
        H = nh * h_tile
        kiota = jax.lax.broadcasted_iota(jnp.int32, (Bt, K), 1)
        lane = jnp.bitwise_and(
            jax.lax.broadcasted_iota(jnp.int32, (Bt, T * C), 1), C - 1)
        cg = cs_ref[...] * C + lane

        def ext_body(k, carry):
            vals, inds = carry
            cv = cm_ref[...]
            m = jnp.max(cv, axis=1, keepdims=True)
            i = jnp.min(jnp.where(cv == m, cg, H), axis=1, keepdims=True)
            cm_ref[...] = jnp.where(cg == i, NEG, cv)
            vals = jnp.where(kiota == k, m, vals)
            inds = jnp.where(kiota == k, i, inds)
            return vals, inds

        vals0 = jnp.zeros((Bt, K), jnp.float32)
        inds0 = jnp.zeros((Bt, K), jnp.int32)
        vals, inds = jax.lax.fori_loop(0, K, ext_body, (vals0, inds0))
        tv_ref[...] = vals
        ti_ref[...] = inds

        cand_left = cm_ref[...]
        used = jnp.where(cand_left == NEG, 1, 0)
        chunk_used = sum(used[:, j * C:(j + 1) * C] for j in range(T))
        saturated = jnp.max(chunk_used, axis=(0, 1)) >= T
        dup = jnp.max(
            jnp.where(vals[:, :-1] == vals[:, 1:], 1, 0), axis=(0, 1)) > 0
        nonpos = jnp.min(vals, axis=(0, 1)) <= 0.0
        mrem = jnp.max(cand_left, axis=1, keepdims=True)
        boundary = jnp.max(
            jnp.where(mrem == vals[:, K - 1:K], 1, 0), axis=(0, 1)) > 0
        flag = saturated | dup | nonpos | boundary
        fl_ref[...] = jnp.zeros((1, 1, 1), jnp.int32) + flag.astype(jnp.int32)


def _enc_topk(x, pre_bias2d, latent_bias2d, W_enc, K, h_tile, b_tile, T=8):
    B, D = x.shape
    H = W_enc.shape[0]
    nh = H // h_tile
    nb = B // b_tile
    C = min(128, h_tile)
    return pl.pallas_call(
        functools.partial(_enc_topk_body, K=K, T=T, nh=nh),
        grid=(nb, nh),
        in_specs=[
            pl.BlockSpec((b_tile, D), lambda b, h: (b, 0)),
            pl.BlockSpec((1, D), lambda b, h: (0, 0)),
            pl.BlockSpec((1, h_tile), lambda b, h: (0, h)),
            pl.BlockSpec((h_tile, D), lambda b, h: (h, 0)),
        ],
        out_specs=[
            pl.BlockSpec((b_tile, h_tile), lambda b, h: (b, h)),
            pl.BlockSpec((b_tile, K), lambda b, h: (b, 0)),
            pl.BlockSpec((b_tile, K), lambda b, h: (b, 0)),
            pl.BlockSpec((1, 1, 1), lambda b, h: (b, 0, 0)),
        ],
        out_shape=[
            jax.ShapeDtypeStruct((B, H), jnp.float32),
            jax.ShapeDtypeStruct((B, K), jnp.float32),
            jax.ShapeDtypeStruct((B, K), jnp.int32),
            jax.ShapeDtypeStruct((nb, 1, 1), jnp.int32),
        ],
        scratch_shapes=[
            pltpu.VMEM((b_tile, T * C), jnp.float32),
            pltpu.VMEM((b_tile, T * C), jnp.int32),
        ],
    )(x, pre_bias2d, latent_bias2d, W_enc)


def _fixup_body(fl_ref, tvin_ref, tiin_ref, pa_any,
                tvout_ref, tiout_ref, buf_ref, work_ref, sem,
                *, K, sub, nsub):
    b = pl.program_id(0)
    Bt = tvin_ref.shape[0]
    H = buf_ref.shape[1]
    tvout_ref[...] = tvin_ref[...]
    tiout_ref[...] = tiin_ref[...]
    flag = fl_ref[0, 0, 0] != 0

    @pl.when(flag)
    def _():
        kiota = jax.lax.broadcasted_iota(jnp.int32, (sub, K), 1)
        iota = jax.lax.broadcasted_iota(jnp.int32, (sub, H), 1)

        def outer(i, _):
            cp = pltpu.make_async_copy(
                pa_any.at[pl.ds(b * Bt + i * sub, sub), :], buf_ref, sem)
            cp.start()
            cp.wait()
            work_ref[...] = jnp.maximum(buf_ref[...], 0.0)

            def body(k, carry):
                svals, sinds = carry
                work = work_ref[...]
                m = jnp.max(work, axis=1, keepdims=True)
                cand = jnp.where(work == m, iota, H)
                idx = jnp.min(cand, axis=1, keepdims=True)
                work_ref[...] = jnp.where(iota == idx, -1.0, work)
                svals = jnp.where(kiota == k, m, svals)
                sinds = jnp.where(kiota == k, idx, sinds)
                return svals, sinds

            svals, sinds = jax.lax.fori_loop(
                0, K, body,
                (jnp.zeros((sub, K), jnp.float32),
                 jnp.zeros((sub, K), jnp.int32)))
            tvout_ref[pl.ds(i * sub, sub), :] = svals
            tiout_ref[pl.ds(i * sub, sub), :] = sinds
            return 0

        jax.lax.fori_loop(0, nsub, outer, 0)


def _fixup(flags, tv, ti, pre_acts, b_tile, sub):
    B, K = tv.shape
    H = pre_acts.shape[1]
    nb = B // b_tile
    return pl.pallas_call(
        functools.partial(_fixup_body, K=K, sub=sub, nsub=b_tile // sub),
        grid=(nb,),
        in_specs=[
            pl.BlockSpec((1, 1, 1), lambda b: (b, 0, 0)),
            pl.BlockSpec((b_tile, K), lambda b: (b, 0)),
            pl.BlockSpec((b_tile, K), lambda b: (b, 0)),
            pl.BlockSpec(memory_space=pl.ANY),
        ],
        out_specs=[
            pl.BlockSpec((b_tile, K), lambda b: (b, 0)),
            pl.BlockSpec((b_tile, K), lambda b: (b, 0)),
        ],
        out_shape=[
            jax.ShapeDtypeStruct((B, K), jnp.float32),
            jax.ShapeDtypeStruct((B, K), jnp.int32),
        ],
        scratch_shapes=[
            pltpu.VMEM((sub, H), jnp.float32),
            pltpu.VMEM((sub, H), jnp.float32),
            pltpu.SemaphoreType.DMA,
        ],
    )(flags, tv, ti, pre_acts)


def _decode_body(pa_ref, wd_ref, pb_ref, mk_ref, ik_ref, sc_ref, out_ref,
                 *, h_tile):
    h = pl.program_id(1)

    pa = pa_ref[...]
    Bt, Ht = pa.shape
    mk = mk_ref[...]
    ik = ik_ref[...]
    gidx = jax.lax.broadcasted_iota(jnp.int32, (Bt, Ht), 1) + h * h_tile
    sel = (pa > mk) | ((pa == mk) & (gidx <= ik))
    sc = jnp.where(sel, pa, 0.0)
    sc_ref[...] = sc

    @pl.when(h == 0)
    def _():
        out_ref[...] = jnp.broadcast_to(pb_ref[...], out_ref.shape)

    out_ref[...] += jax.lax.dot_general(
        sc, wd_ref[...], (((1,), (1,)), ((), ())),
        preferred_element_type=jnp.float32)


def _decode(pre_acts, W_dec, pre_bias2d, mk, ik, h_tile, b_tile):
    B, H = pre_acts.shape
    D = W_dec.shape[0]
    nh = H // h_tile
    nb = B // b_tile
    return pl.pallas_call(
        functools.partial(_decode_body, h_tile=h_tile),
        grid=(nb, nh),
        in_specs=[
            pl.BlockSpec((b_tile, h_tile), lambda b, h: (b, h)),
            pl.BlockSpec((D, h_tile), lambda b, h: (0, h)),
            pl.BlockSpec((1, D), lambda b, h: (0, 0)),
            pl.BlockSpec((b_tile, 1), lambda b, h: (b, 0)),
            pl.BlockSpec((b_tile, 1), lambda b, h: (b, 0)),
        ],
        out_specs=[
            pl.BlockSpec((b_tile, h_tile), lambda b, h: (b, h)),
            pl.BlockSpec((b_tile, D), lambda b, h: (b, 0)),
        ],
        out_shape=[
            jax.ShapeDtypeStruct((B, H), jnp.float32),
            jax.ShapeDtypeStruct((B, D), jnp.float32),
        ],
    )(pre_acts, W_dec, pre_bias2d, mk, ik)


def kernel(x, pre_bias, latent_bias, W_enc, W_dec):
    B, D = x.shape
    H = W_enc.shape[0]
    K = 50
    pb2 = pre_bias.reshape(1, D)
    lb2 = latent_bias.reshape(1, H)

    et_b = min(512, B)
    et_h = min(2048, H)
    pre_acts, tv0, ti0, flags = _enc_topk(
        x, pb2, lb2, W_enc, K, h_tile=et_h, b_tile=et_b)
    topk_values, topk_indices = _fixup(
        flags, tv0, ti0, pre_acts, b_tile=et_b, sub=min(64, B))
    mk = topk_values[:, K - 1:K]
    ik = topk_indices[:, K - 1:K]
    sparse_code, reconstruction = _decode(
        pre_acts, W_dec, pb2, mk, ik,
        h_tile=min(512, H), b_tile=min(2048, B))
    return (reconstruction, sparse_code, pre_acts, topk_values, topk_indices)


# speedup vs baseline: 2.0625x; 2.0625x over previous
import functools

import jax
import jax.numpy as jnp
from jax.experimental import pallas as pl
from jax.experimental.pallas import tpu as pltpu


def _encode_body(x_ref, pb_ref, lb_ref, w_ref, out_ref):
    xc = x_ref[...] - pb_ref[...]
    acc = jax.lax.dot_general(
        xc, w_ref[...], (((1,), (1,)), ((), ())),
        preferred_element_type=jnp.float32)
    out_ref[...] = acc + lb_ref[...]


def _encode(x, pre_bias2d, latent_bias2d, W_enc, h_tile, b_tile):
    B, D = x.shape
    H = W_enc.shape[0]
    nh = H // h_tile
    nb = B // b_tile
    return pl.pallas_call(
        _encode_body,
        grid=(nh, nb),
        in_specs=[
            pl.BlockSpec((b_tile, D), lambda h, b: (b, 0)),
            pl.BlockSpec((1, D), lambda h, b: (0, 0)),
            pl.BlockSpec((1, h_tile), lambda h, b: (0, h)),
            pl.BlockSpec((h_tile, D), lambda h, b: (h, 0)),
        ],
        out_specs=pl.BlockSpec((b_tile, h_tile), lambda h, b: (b, h)),
        out_shape=jax.ShapeDtypeStruct((B, H), jnp.float32),
    )(x, pre_bias2d, latent_bias2d, W_enc)


def _topk_body(pa_ref, tv_ref, ti_ref, work_ref, cv_ref, cg_ref, *, K, T):
    Bt, H = pa_ref.shape
    C = min(128, H)
    S = H // C
    kiota = jax.lax.broadcasted_iota(jnp.int32, (Bt, K), 1)
    NEG = jnp.float32(-jnp.inf)

    mreg = [jnp.full((Bt, C), NEG, jnp.float32) for _ in range(T)]
    sreg = [jnp.full((Bt, C), S, jnp.int32) for _ in range(T)]
    for s in range(S):
        v = pa_ref[:, s * C:(s + 1) * C]
        ci = jnp.full((Bt, C), s, jnp.int32)
        for j in range(T):
            beat = v > mreg[j]
            mo, so = mreg[j], sreg[j]
            mreg[j] = jnp.where(beat, v, mo)
            sreg[j] = jnp.where(beat, ci, so)
            v = jnp.where(beat, mo, v)
            ci = jnp.where(beat, so, ci)
    lane = jax.lax.broadcasted_iota(jnp.int32, (Bt, C), 1)
    for j in range(T):
        cv_ref[:, j * C:(j + 1) * C] = mreg[j]
        cg_ref[:, j * C:(j + 1) * C] = sreg[j] * C + lane

    def ext_body(k, carry):
        vals, inds = carry
        cv = cv_ref[...]
        cg = cg_ref[...]
        m = jnp.max(cv, axis=1, keepdims=True)
        i = jnp.min(jnp.where(cv == m, cg, H), axis=1,
                    keepdims=True)
        cv_ref[...] = jnp.where(cg == i, NEG, cv)
        vals = jnp.where(kiota == k, m, vals)
        inds = jnp.where(kiota == k, i, inds)
        return vals, inds

    vals0 = jnp.zeros((Bt, K), jnp.float32)
    inds0 = jnp.zeros((Bt, K), jnp.int32)
    vals, inds = jax.lax.fori_loop(0, K, ext_body, (vals0, inds0))
    tv_ref[...] = vals
    ti_ref[...] = inds

    cand_left = cv_ref[...]
    used = jnp.where(cand_left == NEG, 1, 0)
    chunk_used = sum(used[:, j * C:(j + 1) * C] for j in range(T))
    saturated = jnp.max(chunk_used, axis=(0, 1)) >= T
    dup = jnp.max(jnp.where(vals[:, :-1] == vals[:, 1:], 1, 0), axis=(0, 1)) > 0
    nonpos = jnp.min(vals, axis=(0, 1)) <= 0.0
    mrem = jnp.max(cand_left, axis=1, keepdims=True)
    boundary = jnp.max(
        jnp.where(mrem == vals[:, K - 1:K], 1, 0), axis=(0, 1)) > 0
    flag = saturated | dup | nonpos | boundary

    @pl.when(flag)
    def _slow():
        iota = jax.lax.broadcasted_iota(jnp.int32, (Bt, H), 1)
        work_ref[...] = jnp.maximum(pa_ref[...], 0.0)

        def body(k, carry):
            svals, sinds = carry
            work = work_ref[...]
            m = jnp.max(work, axis=1, keepdims=True)
            cand = jnp.where(work == m, iota, H)
            idx = jnp.min(cand, axis=1, keepdims=True)
            work_ref[...] = jnp.where(iota == idx, -1.0, work)
            svals = jnp.where(kiota == k, m, svals)
            sinds = jnp.where(kiota == k, idx, sinds)
            return svals, sinds

        svals, sinds = jax.lax.fori_loop(0, K, body, (vals0, inds0))
        tv_ref[...] = svals
        ti_ref[...] = sinds


def _topk(pre_acts, K, b_tile, T=8):
    B, H = pre_acts.shape
    nb = B // b_tile
    return pl.pallas_call(
        functools.partial(_topk_body, K=K, T=T),
        grid=(nb,),
        in_specs=[pl.BlockSpec((b_tile, H), lambda b: (b, 0))],
        out_specs=[
            pl.BlockSpec((b_tile, K), lambda b: (b, 0)),
            pl.BlockSpec((b_tile, K), lambda b: (b, 0)),
        ],
        out_shape=[
            jax.ShapeDtypeStruct((B, K), jnp.float32),
            jax.ShapeDtypeStruct((B, K), jnp.int32),
        ],
        scratch_shapes=[
            pltpu.VMEM((b_tile, H), jnp.float32),
            pltpu.VMEM((b_tile, T * min(128, H)), jnp.float32),
            pltpu.VMEM((b_tile, T * min(128, H)), jnp.int32),
        ],
    )(pre_acts)


def _decode_body(pa_ref, wd_ref, pb_ref, mk_ref, ik_ref, sc_ref, out_ref,
                 *, h_tile):
    h = pl.program_id(1)

    pa = pa_ref[...]
    Bt, Ht = pa.shape
    mk = mk_ref[...]
    ik = ik_ref[...]
    gidx = jax.lax.broadcasted_iota(jnp.int32, (Bt, Ht), 1) + h * h_tile
    sel = (pa > mk) | ((pa == mk) & (gidx <= ik))
    sc = jnp.where(sel, pa, 0.0)
    sc_ref[...] = sc

    @pl.when(h == 0)
    def _():
        out_ref[...] = jnp.broadcast_to(pb_ref[...], out_ref.shape)

    out_ref[...] += jax.lax.dot_general(
        sc, wd_ref[...], (((1,), (1,)), ((), ())),
        preferred_element_type=jnp.float32)


def _decode(pre_acts, W_dec, pre_bias2d, mk, ik, h_tile, b_tile):
    B, H = pre_acts.shape
    D = W_dec.shape[0]
    nh = H // h_tile
    nb = B // b_tile
    return pl.pallas_call(
        functools.partial(_decode_body, h_tile=h_tile),
        grid=(nb, nh),
        in_specs=[
            pl.BlockSpec((b_tile, h_tile), lambda b, h: (b, h)),
            pl.BlockSpec((D, h_tile), lambda b, h: (0, h)),
            pl.BlockSpec((1, D), lambda b, h: (0, 0)),
            pl.BlockSpec((b_tile, 1), lambda b, h: (b, 0)),
            pl.BlockSpec((b_tile, 1), lambda b, h: (b, 0)),
        ],
        out_specs=[
            pl.BlockSpec((b_tile, h_tile), lambda b, h: (b, h)),
            pl.BlockSpec((b_tile, D), lambda b, h: (b, 0)),
        ],
        out_shape=[
            jax.ShapeDtypeStruct((B, H), jnp.float32),
            jax.ShapeDtypeStruct((B, D), jnp.float32),
        ],
    )(pre_acts, W_dec, pre_bias2d, mk, ik)


def kernel(x, pre_bias, latent_bias, W_enc, W_dec):
    B, D = x.shape
    H = W_enc.shape[0]
    K = 50
    pb2 = pre_bias.reshape(1, D)
    lb2 = latent_bias.reshape(1, H)

    pre_acts = _encode(x, pb2, lb2, W_enc, h_tile=2048, b_tile=1024)
    topk_values, topk_indices = _topk(pre_acts, K, b_tile=64)
    mk = topk_values[:, K - 1:K]
    ik = topk_indices[:, K - 1:K]
    sparse_code, reconstruction = _decode(
        pre_acts, W_dec, pb2, mk, ik, h_tile=512, b_tile=2048)
    return (reconstruction, sparse_code, pre_acts, topk_values, topk_indices)
